# baseline (speedup 1.0000x reference)
import functools

import jax
import jax.numpy as jnp
from jax import lax
from jax.experimental import pallas as pl
from jax.experimental.pallas import tpu as pltpu
from jax.experimental.pallas import tpu_sc as plsc

BN_EPS = 1e-5
N = 10000
E = 320000
D = 128
G = 64

NC = 2
NS = 16
NW = NC * NS

CHUNK = 88
NROW = 3
NSI = 4
NCHUNKS = 120
EPW_PAD = NCHUNKS * CHUNK
ROWS_FULL = 640
ROWS_LAST = N - (NS - 1) * ROWS_FULL
DUMP_ROW = N


def _sc_agg(x, src_w, dst_w):
    mesh = plsc.VectorSubcoreMesh(core_axis_name="c", subcore_axis_name="s")

    @functools.partial(
        pl.kernel,
        out_type=jax.ShapeDtypeStruct((NC, N, D), jnp.float32),
        mesh=mesh,
        scratch_types=[
            pltpu.VMEM((NSI, 1, CHUNK), jnp.int32),
            pltpu.VMEM((NCHUNKS, CHUNK), jnp.int32),
            pltpu.VMEM((NROW, CHUNK, D), jnp.float32),
            pltpu.VMEM_SHARED((N + 8, D), jnp.float32),
            pltpu.SemaphoreType.DMA,
            pltpu.SemaphoreType.DMA,
            pltpu.SemaphoreType.DMA,
        ],
    )
    def k(x_hbm, si_hbm, di_hbm, out_hbm, si_v, di_v, rows_v, agg_sh,
          sem_i, sem_g, sem_s):
        c = lax.axis_index("c")
        s = lax.axis_index("s")
        wid = s * NC + c
        r0 = s * ROWS_FULL

        @pl.when(s < NS - 1)
        def _():
            pltpu.sync_copy(x_hbm.at[pl.ds(r0, ROWS_FULL)],
                            agg_sh.at[pl.ds(r0, ROWS_FULL)])

        @pl.when(s == NS - 1)
        def _():
            pltpu.sync_copy(x_hbm.at[pl.ds(r0, ROWS_LAST)],
                            agg_sh.at[pl.ds(r0, ROWS_LAST)])

        pltpu.sync_copy(di_hbm.at[wid], di_v)
        plsc.subcore_barrier()

        def fire_si(j, slot):
            pltpu.async_copy(si_hbm.at[wid, j], si_v.at[slot], sem_i)

        def fire_gather(slot, rslot):
            pltpu.async_copy(x_hbm.at[si_v.at[slot, 0]], rows_v.at[rslot],
                             sem_g)

        def fire_scatter(j, rslot):
            pltpu.async_copy(rows_v.at[rslot], agg_sh.at[di_v.at[j]],
                             sem_s, add=True)

        def drain_rows(sem):
            pltpu.make_async_copy(x_hbm.at[pl.ds(0, CHUNK)],
                                  rows_v.at[0], sem).wait()

        def drain_si():
            pltpu.make_async_copy(si_hbm.at[0, 0], si_v.at[0], sem_i).wait()

        for j in range(NSI):
            fire_si(j, j)
        drain_si()
        fire_gather(0, 0)
        drain_si()
        fire_gather(1, 1)

        @pl.loop(0, NCHUNKS, step=NSI * NROW)
        def _(j0):
            for b in range(NSI * NROW):
                j = j0 + b

                drain_rows(sem_g)
                fire_scatter(j, b % NROW)

                @pl.when(j + NSI < NCHUNKS)
                def _():
                    fire_si(j + NSI, b % NSI)

                @pl.when(j >= 1)
                def _():
                    drain_rows(sem_s)

                @pl.when(j + 2 < NCHUNKS)
                def _():
                    drain_si()
                    fire_gather((b + 2) % NSI, (b + 2) % NROW)

        drain_rows(sem_s)
        plsc.subcore_barrier()

        @pl.when(s < NS - 1)
        def _():
            pltpu.sync_copy(agg_sh.at[pl.ds(r0, ROWS_FULL)],
                            out_hbm.at[c, pl.ds(r0, ROWS_FULL)])

        @pl.when(s == NS - 1)
        def _():
            pltpu.sync_copy(agg_sh.at[pl.ds(r0, ROWS_LAST)],
                            out_hbm.at[c, pl.ds(r0, ROWS_LAST)])

    return k(x, src_w, dst_w)


_BLK = 2000


def _conv_body(x_ref, a_ref, w_ref, b_ref, o_ref):
    h = a_ref[0] + a_ref[1] - x_ref[...]
    y = lax.dot_general(h, w_ref[...], (((1,), (0,)), ((), ())),
                        preferred_element_type=jnp.float32,
                        precision=lax.Precision.HIGHEST)
    o_ref[...] = jnp.maximum(y + b_ref[...], 0.0)


def _conv_tc(x, agg, w, b):
    return pl.pallas_call(
        _conv_body,
        grid=(N // _BLK,),
        in_specs=[
            pl.BlockSpec((_BLK, D), lambda i: (i, 0)),
            pl.BlockSpec((NC, _BLK, D), lambda i: (0, i, 0)),
            pl.BlockSpec((D, D), lambda i: (0, 0)),
            pl.BlockSpec((1, D), lambda i: (0, 0)),
        ],
        out_specs=pl.BlockSpec((_BLK, D), lambda i: (i, 0)),
        out_shape=jax.ShapeDtypeStruct((N, D), jnp.float32),
    )(x, agg, w, b)


_PBLK = 2000
_NPB = N // _PBLK


def _readout_body(b_ref, h1x_ref, h2x_ref, h1y_ref, h2y_ref,
                  w1_ref, b1_ref, w2_ref, b2_ref,
                  o1_ref, o2_ref, acc_ref):
    i = pl.program_id(0)

    @pl.when(i == 0)
    def _():
        acc_ref[...] = jnp.zeros_like(acc_ref)

    bb = b_ref[0, 0, :]
    oh = (lax.broadcasted_iota(jnp.int32, (G, _PBLK), 0)
          == bb[None, :]).astype(jnp.float32)
    hcat = jnp.concatenate(
        [h1x_ref[...], h2x_ref[...], h1y_ref[...], h2y_ref[...]], axis=1)
    acc_ref[...] += lax.dot_general(
        oh, hcat, (((1,), (0,)), ((), ())),
        preferred_element_type=jnp.float32, precision=lax.Precision.HIGHEST)

    @pl.when(i == _NPB - 1)
    def _():
        p = acc_ref[...]
        z = lax.dot_general(p, w1_ref[...], (((1,), (0,)), ((), ())),
                            preferred_element_type=jnp.float32,
                            precision=lax.Precision.HIGHEST)
        z = jnp.maximum(z + b1_ref[...], 0.0)
        z2 = lax.dot_general(z, w2_ref[...], (((1,), (0,)), ((), ())),
                             preferred_element_type=jnp.float32,
                             precision=lax.Precision.HIGHEST)
        z2 = z2 + b2_ref[...]
        o1_ref[...] = z2
        m = jnp.max(z2, axis=1, keepdims=True)
        e = z2 - m
        lse = jnp.log(jnp.sum(jnp.exp(e), axis=1, keepdims=True))
        o2_ref[...] = e - lse


def _readout(batch3, h1x, h2x, h1y, h2y, w1, b1, w2, b2):
    hspec = pl.BlockSpec((_PBLK, D), lambda i: (i, 0))
    return pl.pallas_call(
        _readout_body,
        grid=(_NPB,),
        in_specs=[
            pl.BlockSpec((1, 1, _PBLK), lambda i: (i, 0, 0)),
            hspec, hspec, hspec, hspec,
            pl.BlockSpec((4 * D, 2 * D), lambda i: (0, 0)),
            pl.BlockSpec((1, 2 * D), lambda i: (0, 0)),
            pl.BlockSpec((2 * D, G), lambda i: (0, 0)),
            pl.BlockSpec((1, G), lambda i: (0, 0)),
        ],
        out_specs=[
            pl.BlockSpec((G, G), lambda i: (0, 0)),
            pl.BlockSpec((G, G), lambda i: (0, 0)),
        ],
        out_shape=[
            jax.ShapeDtypeStruct((G, G), jnp.float32),
            jax.ShapeDtypeStruct((G, G), jnp.float32),
        ],
        scratch_shapes=[pltpu.VMEM((G, 4 * D), jnp.float32)],
    )(batch3, h1x, h2x, h1y, h2y, w1, b1, w2, b2)


def _prep_edges(ei):
    src = ei[0].reshape(NW, E // NW)
    dst = ei[1].reshape(NW, E // NW)
    pad = EPW_PAD - E // NW
    src = jnp.pad(src, ((0, 0), (0, pad)))
    dst = jnp.pad(dst, ((0, 0), (0, pad)), constant_values=DUMP_ROW)
    return (src.reshape(NW, NCHUNKS, 1, CHUNK),
            dst.reshape(NW, NCHUNKS, CHUNK))


def _fold_bn(w, b, g, be):
    s = g / jnp.sqrt(1.0 + BN_EPS)
    return w * s[None, :], (b * s + be)[None, :]


def kernel(x, edge_index_x, edge_index_y, batch,
           W1x, b1x, g1x, be1x, W2x, b2x, g2x, be2x,
           W1y, b1y, g1y, be1y, W2y, b2y, g2y, be2y,
           lin1_W, lin1_b, lin2_W, lin2_b):
    sx, dx = _prep_edges(edge_index_x)
    sy, dy = _prep_edges(edge_index_y)
    w1xf, b1xf = _fold_bn(W1x, b1x, g1x, be1x)
    w2xf, b2xf = _fold_bn(W2x, b2x, g2x, be2x)
    w1yf, b1yf = _fold_bn(W1y, b1y, g1y, be1y)
    w2yf, b2yf = _fold_bn(W2y, b2y, g2y, be2y)

    h1x = _conv_tc(x, _sc_agg(x, sx, dx), w1xf, b1xf)
    h1y = _conv_tc(x, _sc_agg(x, sy, dy), w1yf, b1yf)
    h2x = _conv_tc(h1x, _sc_agg(h1x, sx, dx), w2xf, b2xf)
    h2y = _conv_tc(h1y, _sc_agg(h1y, sy, dy), w2yf, b2yf)

    batch3 = batch.reshape(_NPB, 1, _PBLK)
    out, logp = _readout(batch3, h1x, h2x, h1y, h2y,
                         lin1_W, lin1_b.reshape(1, -1),
                         lin2_W, lin2_b.reshape(1, -1))
    return (out, logp)


# speedup vs baseline: 2.7614x; 2.7614x over previous
import functools

import jax
import jax.numpy as jnp
from jax import lax
from jax.experimental import pallas as pl
from jax.experimental.pallas import tpu as pltpu
from jax.experimental.pallas import tpu_sc as plsc

BN_EPS = 1e-5
N = 10000
E = 320000
D = 128
G = 64

NC = 2
NS = 16

CHUNK = 32
NROW = 2
NSI = 4
NCHUNKS = 640
EPW = E // NS
EPW_PAD = NCHUNKS * CHUNK

HALF0 = 5008
HALF1 = N - HALF0
NDUMP = 64
AGG_ROWS = HALF0 + NDUMP
DUMP0 = HALF0

ROWS_FULL = 640
ROWS_LAST = N - (NS - 1) * ROWS_FULL
SC0_FULL = 320
SC0_LAST = HALF0 - (NS - 1) * SC0_FULL
SC1_ROWS = HALF1 // NS


def _sc_agg(src, si, di):
    mesh = plsc.VectorSubcoreMesh(core_axis_name="c", subcore_axis_name="s")

    @functools.partial(
        pl.kernel,
        out_type=jax.ShapeDtypeStruct((N, D), jnp.float32),
        mesh=mesh,
        scratch_types=[
            pltpu.VMEM((NSI, 1, CHUNK), jnp.int32),
            pltpu.VMEM((NSI, 1, CHUNK), jnp.int32),
            pltpu.VMEM((NROW, CHUNK, D), jnp.float32),
            pltpu.VMEM_SHARED((N, D), jnp.float32),
            pltpu.VMEM_SHARED((AGG_ROWS, D), jnp.float32),
            pltpu.SemaphoreType.DMA,
            pltpu.SemaphoreType.DMA,
            pltpu.SemaphoreType.DMA,
            pltpu.SemaphoreType.DMA,
        ],
    )
    def k(src_hbm, si_hbm, di_hbm, out_hbm, si_v, di_v, rows_v,
          xc_sh, agg_sh, sem_i, sem_d, sem_g, sem_s):
        c = lax.axis_index("c")
        s = lax.axis_index("s")
        r0 = s * ROWS_FULL

        @pl.when(s < NS - 1)
        def _():
            pltpu.sync_copy(src_hbm.at[pl.ds(r0, ROWS_FULL)],
                            xc_sh.at[pl.ds(r0, ROWS_FULL)])

        @pl.when(s == NS - 1)
        def _():
            pltpu.sync_copy(src_hbm.at[pl.ds(r0, ROWS_LAST)],
                            xc_sh.at[pl.ds(r0, ROWS_LAST)])

        @pl.when(jnp.logical_and(c == 0, s < NS - 1))
        def _():
            pltpu.sync_copy(src_hbm.at[pl.ds(s * SC0_FULL, SC0_FULL)],
                            agg_sh.at[pl.ds(s * SC0_FULL, SC0_FULL)])

        @pl.when(jnp.logical_and(c == 0, s == NS - 1))
        def _():
            pltpu.sync_copy(
                src_hbm.at[pl.ds((NS - 1) * SC0_FULL, SC0_LAST)],
                agg_sh.at[pl.ds((NS - 1) * SC0_FULL, SC0_LAST)])

        @pl.when(c == 1)
        def _():
            pltpu.sync_copy(
                src_hbm.at[pl.ds(HALF0 + s * SC1_ROWS, SC1_ROWS)],
                agg_sh.at[pl.ds(s * SC1_ROWS, SC1_ROWS)])

        plsc.subcore_barrier()

        def fire_si(j, slot):
            pltpu.async_copy(si_hbm.at[s, j], si_v.at[slot], sem_i)

        def fire_di(j, slot):
            pltpu.async_copy(di_hbm.at[c, s, j], di_v.at[slot], sem_d)

        def fire_gather(slot, rslot):
            pltpu.async_copy(xc_sh.at[si_v.at[slot, 0]], rows_v.at[rslot],
                             sem_g)

        def fire_scatter(slot, rslot):
            pltpu.async_copy(rows_v.at[rslot], agg_sh.at[di_v.at[slot, 0]],
                             sem_s, add=True)

        def drain_rows(sem):
            pltpu.make_async_copy(xc_sh.at[pl.ds(0, CHUNK)],
                                  rows_v.at[0], sem).wait()

        def drain_idx(sem):
            pltpu.make_async_copy(si_hbm.at[0, 0], si_v.at[0], sem).wait()

        for j in range(NSI):
            fire_si(j, j)
            fire_di(j, j)
        drain_idx(sem_i)
        fire_gather(0, 0)
        drain_idx(sem_i)
        fire_gather(1, 1)

        @pl.loop(0, NCHUNKS, step=NSI)
        def _(j0):
            for b in range(NSI):
                j = j0 + b

                drain_rows(sem_g)
                drain_idx(sem_d)
                fire_scatter(b, b % NROW)

                @pl.when(j + NSI < NCHUNKS)
                def _():
                    fire_si(j + NSI, b)

                drain_rows(sem_s)

                @pl.when(j + NSI < NCHUNKS)
                def _():
                    fire_di(j + NSI, b)

                @pl.when(j + 2 < NCHUNKS)
                def _():
                    drain_idx(sem_i)
                    fire_gather((b + 2) % NSI, b % NROW)

        plsc.subcore_barrier()

        @pl.when(jnp.logical_and(c == 0, s < NS - 1))
        def _():
            pltpu.sync_copy(agg_sh.at[pl.ds(s * SC0_FULL, SC0_FULL)],
                            out_hbm.at[pl.ds(s * SC0_FULL, SC0_FULL)])

        @pl.when(jnp.logical_and(c == 0, s == NS - 1))
        def _():
            pltpu.sync_copy(
                agg_sh.at[pl.ds((NS - 1) * SC0_FULL, SC0_LAST)],
                out_hbm.at[pl.ds((NS - 1) * SC0_FULL, SC0_LAST)])

        @pl.when(c == 1)
        def _():
            pltpu.sync_copy(
                agg_sh.at[pl.ds(s * SC1_ROWS, SC1_ROWS)],
                out_hbm.at[pl.ds(HALF0 + s * SC1_ROWS, SC1_ROWS)])

    return k(src, si, di)


_BLK = 2000


def _conv_body(a_ref, w_ref, b_ref, o_ref):
    y = lax.dot_general(a_ref[...], w_ref[...], (((1,), (0,)), ((), ())),
                        preferred_element_type=jnp.float32,
                        precision=lax.Precision.HIGHEST)
    o_ref[...] = jnp.maximum(y + b_ref[...], 0.0)


def _conv_tc(agg, w, b):
    return pl.pallas_call(
        _conv_body,
        grid=(N // _BLK,),
        in_specs=[
            pl.BlockSpec((_BLK, D), lambda i: (i, 0)),
            pl.BlockSpec((D, D), lambda i: (0, 0)),
            pl.BlockSpec((1, D), lambda i: (0, 0)),
        ],
        out_specs=pl.BlockSpec((_BLK, D), lambda i: (i, 0)),
        out_shape=jax.ShapeDtypeStruct((N, D), jnp.float32),
    )(agg, w, b)


_PBLK = 2000
_NPB = N // _PBLK


def _readout_body(b_ref, h1x_ref, h2x_ref, h1y_ref, h2y_ref,
                  w1_ref, b1_ref, w2_ref, b2_ref,
                  o1_ref, o2_ref, acc_ref):
    i = pl.program_id(0)

    @pl.when(i == 0)
    def _():
        acc_ref[...] = jnp.zeros_like(acc_ref)

    bb = b_ref[0, 0, :]
    oh = (lax.broadcasted_iota(jnp.int32, (G, _PBLK), 0)
          == bb[None, :]).astype(jnp.float32)
    hcat = jnp.concatenate(
        [h1x_ref[...], h2x_ref[...], h1y_ref[...], h2y_ref[...]], axis=1)
    acc_ref[...] += lax.dot_general(
        oh, hcat, (((1,), (0,)), ((), ())),
        preferred_element_type=jnp.float32, precision=lax.Precision.HIGHEST)

    @pl.when(i == _NPB - 1)
    def _():
        p = acc_ref[...]
        z = lax.dot_general(p, w1_ref[...], (((1,), (0,)), ((), ())),
                            preferred_element_type=jnp.float32,
                            precision=lax.Precision.HIGHEST)
        z = jnp.maximum(z + b1_ref[...], 0.0)
        z2 = lax.dot_general(z, w2_ref[...], (((1,), (0,)), ((), ())),
                             preferred_element_type=jnp.float32,
                             precision=lax.Precision.HIGHEST)
        z2 = z2 + b2_ref[...]
        o1_ref[...] = z2
        m = jnp.max(z2, axis=1, keepdims=True)
        e = z2 - m
        lse = jnp.log(jnp.sum(jnp.exp(e), axis=1, keepdims=True))
        o2_ref[...] = e - lse


def _readout(batch3, h1x, h2x, h1y, h2y, w1, b1, w2, b2):
    hspec = pl.BlockSpec((_PBLK, D), lambda i: (i, 0))
    return pl.pallas_call(
        _readout_body,
        grid=(_NPB,),
        in_specs=[
            pl.BlockSpec((1, 1, _PBLK), lambda i: (i, 0, 0)),
            hspec, hspec, hspec, hspec,
            pl.BlockSpec((4 * D, 2 * D), lambda i: (0, 0)),
            pl.BlockSpec((1, 2 * D), lambda i: (0, 0)),
            pl.BlockSpec((2 * D, G), lambda i: (0, 0)),
            pl.BlockSpec((1, G), lambda i: (0, 0)),
        ],
        out_specs=[
            pl.BlockSpec((G, G), lambda i: (0, 0)),
            pl.BlockSpec((G, G), lambda i: (0, 0)),
        ],
        out_shape=[
            jax.ShapeDtypeStruct((G, G), jnp.float32),
            jax.ShapeDtypeStruct((G, G), jnp.float32),
        ],
        scratch_shapes=[pltpu.VMEM((G, 4 * D), jnp.float32)],
    )(batch3, h1x, h2x, h1y, h2y, w1, b1, w2, b2)


def _prep_edges(ei):
    src = ei[0]
    dst = ei[1]
    spread = DUMP0 + (jnp.arange(E, dtype=jnp.int32) % NDUMP)
    d0 = jnp.where(dst < HALF0, dst, spread)
    d1 = jnp.where(dst >= HALF0, dst - HALF0, spread)

    def shape(row, fill):
        r = row.reshape(NS, EPW)
        r = jnp.pad(r, ((0, 0), (0, EPW_PAD - EPW)), constant_values=fill)
        return r.reshape(NS, NCHUNKS, 1, CHUNK)

    si = shape(src, 0)
    di = jnp.stack([shape(d0, DUMP0), shape(d1, DUMP0)])
    return si, di


def _fold_bn(w, b, g, be):
    s = g / jnp.sqrt(1.0 + BN_EPS)
    return w * s[None, :], (b * s + be)[None, :]


def kernel(x, edge_index_x, edge_index_y, batch,
           W1x, b1x, g1x, be1x, W2x, b2x, g2x, be2x,
           W1y, b1y, g1y, be1y, W2y, b2y, g2y, be2y,
           lin1_W, lin1_b, lin2_W, lin2_b):
    six, dix = _prep_edges(edge_index_x)
    siy, diy = _prep_edges(edge_index_y)
    w1xf, b1xf = _fold_bn(W1x, b1x, g1x, be1x)
    w2xf, b2xf = _fold_bn(W2x, b2x, g2x, be2x)
    w1yf, b1yf = _fold_bn(W1y, b1y, g1y, be1y)
    w2yf, b2yf = _fold_bn(W2y, b2y, g2y, be2y)

    h1x = _conv_tc(_sc_agg(x, six, dix), w1xf, b1xf)
    h1y = _conv_tc(_sc_agg(x, siy, diy), w1yf, b1yf)
    h2x = _conv_tc(_sc_agg(h1x, six, dix), w2xf, b2xf)
    h2y = _conv_tc(_sc_agg(h1y, siy, diy), w2yf, b2yf)

    batch3 = batch.reshape(_NPB, 1, _PBLK)
    out, logp = _readout(batch3, h1x, h2x, h1y, h2y,
                         lin1_W, lin1_b.reshape(1, -1),
                         lin2_W, lin2_b.reshape(1, -1))
    return (out, logp)
